# Initial kernel scaffold; baseline (speedup 1.0000x reference)
import jax
import jax.numpy as jnp
from jax.experimental import pallas as pl


def kernel(x, emb_weight):
    raise NotImplementedError("write your pallas kernel here")


# speedup vs baseline: 1.2877x; 1.2877x over previous
import functools

import jax
import jax.numpy as jnp
from jax import lax
from jax.experimental import pallas as pl
from jax.experimental.pallas import tpu as pltpu
from jax.experimental.pallas import tpu_sc as plsc

VOCAB_LOCAL = 100000
N_EMBD = 1024
NUM_TOKENS = 8192

NUM_CORES = 2
NUM_SUBCORES = 16
LANES = 16
NUM_WORKERS = NUM_CORES * NUM_SUBCORES
TOKENS_PER_WORKER = NUM_TOKENS // NUM_WORKERS
CHUNK = 32
NUM_CHUNKS = TOKENS_PER_WORKER // CHUNK
NBUF = 3


@functools.partial(
    pl.kernel,
    mesh=plsc.VectorSubcoreMesh(core_axis_name="c", subcore_axis_name="s"),
    out_type=jax.ShapeDtypeStruct((NUM_TOKENS, N_EMBD), jnp.float32),
    scratch_types=[
        pltpu.VMEM((NUM_CHUNKS, CHUNK), jnp.int32),
        pltpu.VMEM((CHUNK, N_EMBD), jnp.float32),
        pltpu.VMEM((CHUNK, N_EMBD), jnp.float32),
        pltpu.VMEM((CHUNK, N_EMBD), jnp.float32),
        pltpu.SemaphoreType.DMA,
        pltpu.SemaphoreType.DMA,
        pltpu.SemaphoreType.DMA,
        pltpu.SemaphoreType.DMA,
        pltpu.SemaphoreType.DMA,
        pltpu.SemaphoreType.DMA,
    ],
)
def _emb_double(table_hbm, x_hbm, out_hbm, idx_v, b0, b1, b2,
                g0, g1, g2, s0, s1, s2):
    bufs = (b0, b1, b2)
    gsems = (g0, g1, g2)
    ssems = (s0, s1, s2)

    wid = lax.axis_index("s") * NUM_CORES + lax.axis_index("c")
    row_base = wid * TOKENS_PER_WORKER

    pltpu.sync_copy(x_hbm.at[wid], idx_v)

    def start_gather(k):
        b = k % NBUF
        return pltpu.async_copy(table_hbm.at[idx_v.at[k]], bufs[b], gsems[b])

    def start_scatter(k):
        b = k % NBUF
        return pltpu.async_copy(
            bufs[b], out_hbm.at[pl.ds(row_base + k * CHUNK, CHUNK)], ssems[b])

    def double_in_place(b):
        buf = bufs[b]

        def row_body(r, carry):
            for j in range(N_EMBD // LANES):
                v = buf[r, pl.ds(j * LANES, LANES)]
                buf[r, pl.ds(j * LANES, LANES)] = v + v
            return carry

        lax.fori_loop(0, CHUNK, row_body, 0)

    gh = {}
    sh = {}
    waited = set()
    for k in range(min(NBUF - 1, NUM_CHUNKS)):
        gh[k] = start_gather(k)

    for k in range(NUM_CHUNKS):
        nk = k + NBUF - 1
        if nk < NUM_CHUNKS:
            if k >= 1:
                sh[k - 1].wait()
                waited.add(k - 1)
            gh[nk] = start_gather(nk)
        gh[k].wait()
        double_in_place(k % NBUF)
        sh[k] = start_scatter(k)

    for k in range(NUM_CHUNKS):
        if k not in waited:
            sh[k].wait()


def kernel(x, emb_weight):
    x = x.astype(jnp.int32).reshape(NUM_WORKERS, NUM_CHUNKS, CHUNK)
    return _emb_double(emb_weight, x)


# speedup vs baseline: 1.3281x; 1.0313x over previous
import functools

import jax
import jax.numpy as jnp
from jax import lax
from jax.experimental import pallas as pl
from jax.experimental.pallas import tpu as pltpu
from jax.experimental.pallas import tpu_sc as plsc

VOCAB_LOCAL = 100000
N_EMBD = 1024
NUM_TOKENS = 8192

NUM_CORES = 2
NUM_SUBCORES = 16
LANES = 16
NUM_WORKERS = NUM_CORES * NUM_SUBCORES
TOKENS_PER_WORKER = NUM_TOKENS // NUM_WORKERS
CHUNK = 32
HALF = CHUNK // 2
NUM_CHUNKS = TOKENS_PER_WORKER // CHUNK
NBUF = 3


@functools.partial(
    pl.kernel,
    mesh=plsc.VectorSubcoreMesh(core_axis_name="c", subcore_axis_name="s"),
    out_type=jax.ShapeDtypeStruct((NUM_TOKENS, N_EMBD), jnp.float32),
    scratch_types=[
        pltpu.VMEM((TOKENS_PER_WORKER,), jnp.int32),
        pltpu.VMEM((CHUNK, N_EMBD), jnp.float32),
        pltpu.VMEM((CHUNK, N_EMBD), jnp.float32),
        pltpu.VMEM((CHUNK, N_EMBD), jnp.float32),
        pltpu.SemaphoreType.DMA,
        pltpu.SemaphoreType.DMA,
        pltpu.SemaphoreType.DMA,
        pltpu.SemaphoreType.DMA,
        pltpu.SemaphoreType.DMA,
        pltpu.SemaphoreType.DMA,
    ],
)
def _emb_double(table_hbm, x_hbm, out_hbm, idx_v, b0, b1, b2,
                g0, g1, g2, s0, s1, s2):
    bufs = (b0, b1, b2)
    gsems = (g0, g1, g2)
    ssems = (s0, s1, s2)

    wid = lax.axis_index("s") * NUM_CORES + lax.axis_index("c")
    row_base = wid * TOKENS_PER_WORKER

    pltpu.sync_copy(x_hbm.at[pl.ds(row_base, TOKENS_PER_WORKER)], idx_v)

    def start_gather(k):
        b = k % NBUF
        return pltpu.async_copy(
            table_hbm.at[idx_v.at[pl.ds(k * CHUNK, CHUNK)]], bufs[b], gsems[b])

    def start_scatter_half(k, h):
        b = k % NBUF
        return pltpu.async_copy(
            bufs[b].at[pl.ds(h * HALF, HALF)],
            out_hbm.at[pl.ds(row_base + k * CHUNK + h * HALF, HALF)],
            ssems[b])

    def double_half(b, h):
        buf = bufs[b]

        def row_body(r, carry):
            for j in range(N_EMBD // LANES):
                v = buf[r, pl.ds(j * LANES, LANES)]
                buf[r, pl.ds(j * LANES, LANES)] = v + v
            return carry

        lax.fori_loop(h * HALF, (h + 1) * HALF, row_body, 0)

    gh = {}
    sh = {}
    waited = set()
    for k in range(min(NBUF - 1, NUM_CHUNKS)):
        gh[k] = start_gather(k)

    for k in range(NUM_CHUNKS):
        nk = k + NBUF - 1
        if nk < NUM_CHUNKS:
            if k >= 1:
                sh[k - 1][0].wait()
                sh[k - 1][1].wait()
                waited.add(k - 1)
            gh[nk] = start_gather(nk)
        gh[k].wait()
        b = k % NBUF
        double_half(b, 0)
        h0 = start_scatter_half(k, 0)
        double_half(b, 1)
        h1 = start_scatter_half(k, 1)
        sh[k] = (h0, h1)

    for k in range(NUM_CHUNKS):
        if k not in waited:
            sh[k][0].wait()
            sh[k][1].wait()


def kernel(x, emb_weight):
    return _emb_double(emb_weight, x.astype(jnp.int32))
